# baseline (speedup 1.0000x reference)
import jax
import jax.numpy as jnp
from jax import lax
from jax.experimental import pallas as pl
from jax.experimental.pallas import tpu as pltpu
from jax.experimental.pallas import tpu_sc as plsc

_L = 16
_N_RES = 256
_C_S = 384
_N_ATOM = 1024
_C_OUT = 2
_NC = 2
_NS = 16
_ATOMS_PER_W = _N_ATOM // (_NC * _NS)


def _allsum(v, lane):
    for sh in (1, 2, 4, 8):
        v = v + v.at[lane ^ sh].get(mode="promise_in_bounds")
    return v


def _sc_body(s_hbm, t_hbm, w_hbm, b_hbm, out_hbm,
             s_vmem, w_vmem, b_vmem, t_vmem, sw_tile, sw_full, out_tile,
             sw_shared, sem, sem2, sem3, sem4, sem5):
    cid = lax.axis_index("c")
    sid = lax.axis_index("s")
    wid = sid * _NC + cid

    lane = lax.iota(jnp.int32, _L)

    h = _L // 2
    s_copy_lo = pltpu.async_copy(
        s_hbm.at[0, pl.ds(sid * _L, h), :], s_vmem.at[pl.ds(0, h), :], sem2)
    w_copy = pltpu.async_copy(w_hbm, w_vmem, sem3)
    s_copy_hi = pltpu.async_copy(
        s_hbm.at[0, pl.ds(sid * _L + h, h), :], s_vmem.at[pl.ds(h, h), :],
        sem5)
    b_copy = pltpu.async_copy(b_hbm, b_vmem.at[pl.ds(0, _C_OUT)], sem4)
    t_copy = pltpu.async_copy(
        t_hbm.at[0, pl.ds(wid * _ATOMS_PER_W, _ATOMS_PER_W), :], t_vmem, sem)
    s_copy_lo.wait()
    w_copy.wait()
    b_copy.wait()

    nk = _C_S // _L

    def p1_kblock_rows(j0, j1):
        def body(k, accs):
            w0 = w_vmem[0, pl.ds(k * _L, _L)]
            w1 = w_vmem[1, pl.ds(k * _L, _L)]
            new = []
            for i, j in enumerate(range(j0, j1)):
                sv = s_vmem[j, pl.ds(k * _L, _L)]
                new.append(accs[2 * i] + sv * w0)
                new.append(accs[2 * i + 1] + sv * w1)
            return tuple(new)
        return body

    z = jnp.zeros((_L,), jnp.float32)
    accs_lo = lax.fori_loop(0, nk, p1_kblock_rows(0, h), (z,) * (2 * h))
    s_copy_hi.wait()
    accs_hi = lax.fori_loop(0, nk, p1_kblock_rows(h, _L), (z,) * (2 * h))
    accs = accs_lo + accs_hi
    packed = [z, z]
    for j in range(_L):
        sel = lane == j
        for c in range(_C_OUT):
            tot = _allsum(accs[2 * j + c], lane)
            packed[c] = jnp.where(sel, tot, packed[c])
    for c in range(_C_OUT):
        sw_tile[pl.ds(c * _L, _L)] = packed[c]
    pltpu.sync_copy(sw_tile, sw_shared.at[pl.ds(sid * 2 * _L, 2 * _L)])
    plsc.subcore_barrier()
    pltpu.sync_copy(sw_shared, sw_full)

    bvec = b_vmem[pl.ds(0, _L)]
    bias_pair = bvec.at[lane & 1].get(mode="promise_in_bounds")

    t_copy.wait()
    nr = _N_RES // _L

    z = jnp.zeros((_L,), jnp.float32)

    def p2_group(g, _):
        def p2_kblock(k, accs):
            sw0 = sw_full[pl.ds(k * 2 * _L, _L)]
            sw1 = sw_full[pl.ds(k * 2 * _L + _L, _L)]
            new = []
            for a_loc in range(8):
                tv = t_vmem[g * 8 + a_loc, pl.ds(k * _L, _L)]
                new.append(accs[2 * a_loc] + tv * sw0)
                new.append(accs[2 * a_loc + 1] + tv * sw1)
            return tuple(new)

        accs = lax.fori_loop(0, nr, p2_kblock, (z,) * 16)
        packed = z
        for a_loc in range(8):
            for c in range(_C_OUT):
                tot = _allsum(accs[2 * a_loc + c], lane)
                packed = jnp.where(lane == (a_loc * _C_OUT + c), tot, packed)
        out_tile[pl.ds(g * _L, _L)] = packed + bias_pair
        return 0

    lax.fori_loop(0, _ATOMS_PER_W // 8, p2_group, 0)
    pltpu.sync_copy(
        out_tile,
        out_hbm.at[pl.ds(wid * _ATOMS_PER_W * _C_OUT, _ATOMS_PER_W * _C_OUT)])


def kernel(s, token_to_atom_idx, W, b):
    B, n_res, c_s = s.shape
    _, n_atom, _ = token_to_atom_idx.shape
    c_out = W.shape[0]
    f = pl.kernel(
        _sc_body,
        out_type=jax.ShapeDtypeStruct((n_atom * c_out,), jnp.float32),
        mesh=plsc.VectorSubcoreMesh(core_axis_name="c", subcore_axis_name="s"),
        scratch_types=[
            pltpu.VMEM((_L, _C_S), jnp.float32),
            pltpu.VMEM((_C_OUT, _C_S), jnp.float32),
            pltpu.VMEM((_L,), jnp.float32),
            pltpu.VMEM((_ATOMS_PER_W, _N_RES), jnp.float32),
            pltpu.VMEM((_C_OUT * _L,), jnp.float32),
            pltpu.VMEM((_C_OUT * _N_RES,), jnp.float32),
            pltpu.VMEM((_ATOMS_PER_W * _C_OUT,), jnp.float32),
            pltpu.VMEM_SHARED((_C_OUT * _N_RES,), jnp.float32),
            pltpu.SemaphoreType.DMA,
            pltpu.SemaphoreType.DMA,
            pltpu.SemaphoreType.DMA,
            pltpu.SemaphoreType.DMA,
            pltpu.SemaphoreType.DMA,
        ],
    )
    out = f(s, token_to_atom_idx, W, b)
    return out.reshape(B, n_atom, c_out)


# speedup vs baseline: 1.0085x; 1.0085x over previous
import jax
import jax.numpy as jnp
from jax import lax
from jax.experimental import pallas as pl
from jax.experimental.pallas import tpu as pltpu
from jax.experimental.pallas import tpu_sc as plsc

_L = 16
_N_RES = 256
_C_S = 384
_N_ATOM = 1024
_C_OUT = 2
_NC = 2
_NS = 16
_ATOMS_PER_W = _N_ATOM // (_NC * _NS)


def _allsum(v, lane):
    for sh in (1, 2, 4, 8):
        v = v + v.at[lane ^ sh].get(mode="promise_in_bounds")
    return v


def _sc_body(s_hbm, t_hbm, w_hbm, b_hbm, out_hbm,
             s_vmem, w_vmem, b_vmem, t_vmem, sw_tile, sw_full, out_tile,
             sw_shared, sem, sem2, sem3, sem4, sem5):
    cid = lax.axis_index("c")
    sid = lax.axis_index("s")
    wid = sid * _NC + cid

    lane = lax.iota(jnp.int32, _L)

    s_copy = pltpu.async_copy(
        s_hbm.at[0, pl.ds(sid * _L, _L), :], s_vmem, sem2)
    w_copy = pltpu.async_copy(w_hbm, w_vmem, sem3)
    b_copy = pltpu.async_copy(b_hbm, b_vmem.at[pl.ds(0, _C_OUT)], sem4)
    t_copy = pltpu.async_copy(
        t_hbm.at[0, pl.ds(wid * _ATOMS_PER_W, _ATOMS_PER_W), :], t_vmem, sem)
    s_copy.wait()
    w_copy.wait()
    b_copy.wait()

    nk = _C_S // _L

    def p1_kblock(k, accs):
        w0 = w_vmem[0, pl.ds(k * _L, _L)]
        w1 = w_vmem[1, pl.ds(k * _L, _L)]
        new = []
        for j in range(_L):
            sv = s_vmem[j, pl.ds(k * _L, _L)]
            new.append(accs[2 * j] + sv * w0)
            new.append(accs[2 * j + 1] + sv * w1)
        return tuple(new)

    z = jnp.zeros((_L,), jnp.float32)
    accs = lax.fori_loop(0, nk, p1_kblock, (z,) * (2 * _L))
    packed = [z, z]
    for j in range(_L):
        sel = lane == j
        for c in range(_C_OUT):
            tot = _allsum(accs[2 * j + c], lane)
            packed[c] = jnp.where(sel, tot, packed[c])
    for c in range(_C_OUT):
        sw_tile[pl.ds(c * _L, _L)] = packed[c]
    pltpu.sync_copy(sw_tile, sw_shared.at[pl.ds(sid * 2 * _L, 2 * _L)])
    plsc.subcore_barrier()
    pltpu.sync_copy(sw_shared, sw_full)

    bvec = b_vmem[pl.ds(0, _L)]
    bias_pair = bvec.at[lane & 1].get(mode="promise_in_bounds")

    t_copy.wait()
    nr = _N_RES // _L

    z = jnp.zeros((_L,), jnp.float32)

    def p2_group(g, _):
        def p2_kblock(k, accs):
            sw0 = sw_full[pl.ds(k * 2 * _L, _L)]
            sw1 = sw_full[pl.ds(k * 2 * _L + _L, _L)]
            new = []
            for a_loc in range(8):
                tv = t_vmem[g * 8 + a_loc, pl.ds(k * _L, _L)]
                new.append(accs[2 * a_loc] + tv * sw0)
                new.append(accs[2 * a_loc + 1] + tv * sw1)
            return tuple(new)

        accs = lax.fori_loop(0, nr, p2_kblock, (z,) * 16)
        packed = z
        for a_loc in range(8):
            for c in range(_C_OUT):
                tot = _allsum(accs[2 * a_loc + c], lane)
                packed = jnp.where(lane == (a_loc * _C_OUT + c), tot, packed)
        out_tile[pl.ds(g * _L, _L)] = packed + bias_pair
        return 0

    lax.fori_loop(0, _ATOMS_PER_W // 8, p2_group, 0)
    pltpu.sync_copy(
        out_tile,
        out_hbm.at[pl.ds(wid * _ATOMS_PER_W * _C_OUT, _ATOMS_PER_W * _C_OUT)])


def kernel(s, token_to_atom_idx, W, b):
    B, n_res, c_s = s.shape
    _, n_atom, _ = token_to_atom_idx.shape
    c_out = W.shape[0]
    f = pl.kernel(
        _sc_body,
        out_type=jax.ShapeDtypeStruct((n_atom * c_out,), jnp.float32),
        mesh=plsc.VectorSubcoreMesh(core_axis_name="c", subcore_axis_name="s"),
        scratch_types=[
            pltpu.VMEM((_L, _C_S), jnp.float32),
            pltpu.VMEM((_C_OUT, _C_S), jnp.float32),
            pltpu.VMEM((_L,), jnp.float32),
            pltpu.VMEM((_ATOMS_PER_W, _N_RES), jnp.float32),
            pltpu.VMEM((_C_OUT * _L,), jnp.float32),
            pltpu.VMEM((_C_OUT * _N_RES,), jnp.float32),
            pltpu.VMEM((_ATOMS_PER_W * _C_OUT,), jnp.float32),
            pltpu.VMEM_SHARED((_C_OUT * _N_RES,), jnp.float32),
            pltpu.SemaphoreType.DMA,
            pltpu.SemaphoreType.DMA,
            pltpu.SemaphoreType.DMA,
            pltpu.SemaphoreType.DMA,
            pltpu.SemaphoreType.DMA,
        ],
    )
    out = f(s, token_to_atom_idx, W, b)
    return out.reshape(B, n_atom, c_out)
